# baseline (speedup 1.0000x reference)
import jax
import jax.numpy as jnp
from jax import lax
from jax.experimental import pallas as pl
from jax.experimental.pallas import tpu as pltpu
from jax.experimental.pallas import tpu_sc as plsc

N = 10000
N2 = 10240
E = 160000
D_IN, D_H, D_OUT = 256, 512, 256
NC, NS = 2, 16
NW = NC * NS
EB = 125
TPT = E // (NW * EB)
NPT = N2 // NS
RB = 1000
GRID = N // RB
NBUF = 2

_mesh = plsc.VectorSubcoreMesh(core_axis_name="c", subcore_axis_name="s")


def _deg_partials(dst3d, ones_eb, zeros_n):
    def body(dst_r, ones_r, zeros_r, out_r, idx_v, ones_v, acc, sem):
        core = lax.axis_index("c")
        sid = lax.axis_index("s")
        wid = core * NS + sid

        @pl.when(sid == 0)
        def _():
            pltpu.sync_copy(zeros_r, acc)

        pltpu.sync_copy(ones_r, ones_v)
        pltpu.sync_copy(dst_r.at[wid], idx_v)
        plsc.subcore_barrier()

        def step(i, carry):
            pltpu.sync_copy(ones_v, acc.at[idx_v.at[i]], add=True)
            return carry

        lax.fori_loop(0, TPT, step, 0)
        plsc.subcore_barrier()

        @pl.when(sid == 0)
        def _():
            pltpu.sync_copy(acc, out_r.at[core, 0])

    f = pl.kernel(
        body,
        out_type=jax.ShapeDtypeStruct((NC, 1, N2), jnp.float32),
        mesh=_mesh,
        scratch_types=[
            pltpu.VMEM((TPT, EB), jnp.int32),
            pltpu.VMEM((EB,), jnp.float32),
            pltpu.VMEM_SHARED((N2,), jnp.float32),
            pltpu.SemaphoreType.DMA,
        ],
    )
    return f(dst3d, ones_eb, zeros_n)


def _segment_sum(y_chunks, src3d, dst3d, zeros_blk):
    C = len(y_chunks)

    def body(*refs):
        ys = refs[:C]
        src_r, dst_r, zeros_r, out_r = refs[C:C + 4]
        src_v, dst_v = refs[C + 4:C + 6]
        rows = refs[C + 6:C + 6 + NBUF]
        acc = refs[C + 6 + NBUF]
        gsem = refs[C + 7 + NBUF:C + 7 + 2 * NBUF]
        ssem = refs[C + 7 + 2 * NBUF:]

        core = lax.axis_index("c")
        sid = lax.axis_index("s")
        wid = core * NS + sid

        pltpu.sync_copy(src_r.at[wid], src_v)
        pltpu.sync_copy(dst_r.at[wid], dst_v)

        for c in range(C):
            y_r = ys[c]

            def gather(i, b):
                pltpu.async_copy(y_r.at[src_v.at[i]], rows[b], gsem[b])

            def gather_wait(i, b):
                pltpu.make_async_copy(y_r.at[src_v.at[i]], rows[b],
                                      gsem[b]).wait()

            def scatter(i, b):
                pltpu.sync_copy(rows[b], acc.at[dst_v.at[i]], add=True)

            def scatter_wait(i, b):
                del i, b

            pltpu.sync_copy(zeros_r, acc.at[pl.ds(sid * NPT, NPT)])
            plsc.subcore_barrier()

            gather(0, 0)

            def step(t, carry):
                for b in range(NBUF):
                    i = NBUF * t + b
                    gather_wait(i, b)
                    scatter(i, b)
                    nb = (b + 1) % NBUF
                    if b < NBUF - 1:
                        @pl.when(t > 0)
                        def _():
                            scatter_wait(i - (NBUF - 1), nb)

                        gather(i + 1, nb)
                    else:
                        @pl.when(t < TPT // NBUF - 1)
                        def _():
                            scatter_wait(i - (NBUF - 1), nb)
                            gather(i + 1, nb)
                return carry

            lax.fori_loop(0, TPT // NBUF, step, 0)
            for b in range(NBUF):
                scatter_wait(TPT - NBUF + b, b)
            plsc.subcore_barrier()
            pltpu.sync_copy(acc.at[pl.ds(sid * NPT, NPT)],
                            out_r.at[core, c, pl.ds(sid * NPT, NPT)])
            plsc.subcore_barrier()

    f = pl.kernel(
        body,
        out_type=jax.ShapeDtypeStruct((NC, C, N2, 128), jnp.float32),
        mesh=_mesh,
        scratch_types=(
            [pltpu.VMEM((TPT, EB), jnp.int32)] * 2
            + [pltpu.VMEM((EB, 128), jnp.float32)] * NBUF
            + [pltpu.VMEM_SHARED((N2, 128), jnp.float32)]
            + [pltpu.SemaphoreType.DMA] * (2 * NBUF)
        ),
    )
    return f(*y_chunks, src3d, dst3d, zeros_blk)


def _dinv_of(d):
    return lax.rsqrt(d[:, 0:1] + d[:, 1:2] + 1.0)


def _stage1(x, w_lin, w1, deg_t):
    def body(x_r, wl_r, w1_r, d_r, *out_rs):
        wc_v = out_rs[-1]
        out_rs = out_rs[:-1]

        @pl.when(pl.program_id(0) == 0)
        def _():
            wc_v[...] = jnp.dot(wl_r[...], w1_r[...],
                                preferred_element_type=jnp.float32)

        dinv = _dinv_of(d_r[...])
        y = jnp.dot(x_r[...], wc_v[...],
                    preferred_element_type=jnp.float32) * dinv
        for c in range(4):
            out_rs[c][...] = y[:, c * 128:(c + 1) * 128]

    return pl.pallas_call(
        body,
        grid=(GRID,),
        in_specs=[
            pl.BlockSpec((RB, D_IN), lambda i: (i, 0)),
            pl.BlockSpec((D_IN, D_IN), lambda i: (0, 0)),
            pl.BlockSpec((D_IN, D_H), lambda i: (0, 0)),
            pl.BlockSpec((RB, NC), lambda i: (i, 0)),
        ],
        out_specs=[pl.BlockSpec((RB, 128), lambda i: (i, 0))] * 4,
        out_shape=[jax.ShapeDtypeStruct((N, 128), jnp.float32)] * 4,
        scratch_shapes=[pltpu.VMEM((D_IN, D_H), jnp.float32)],
    )(x, w_lin, w1, deg_t)


def _stage2(agg1, y1_chunks, w2r, b1r, deg_t):
    def body(a_r, y0, y1, y2, y3, w_r, b_r, d_r, o0, o1):
        dinv = _dinv_of(d_r[...])
        yc = (y0, y1, y2, y3)
        acc = jnp.zeros((RB, D_OUT), jnp.float32)
        for c in range(4):
            t = a_r[0, c] + a_r[1, c] + yc[c][...]
            h = jnp.maximum(t * dinv + b_r[c][None, :], 0.0)
            acc = acc + jnp.dot(h, w_r[c],
                                preferred_element_type=jnp.float32)
        y2o = acc * dinv
        o0[...] = y2o[:, :128]
        o1[...] = y2o[:, 128:]

    return pl.pallas_call(
        body,
        grid=(GRID,),
        in_specs=[
            pl.BlockSpec((NC, 4, RB, 128), lambda i: (0, 0, i, 0)),
            pl.BlockSpec((RB, 128), lambda i: (i, 0)),
            pl.BlockSpec((RB, 128), lambda i: (i, 0)),
            pl.BlockSpec((RB, 128), lambda i: (i, 0)),
            pl.BlockSpec((RB, 128), lambda i: (i, 0)),
            pl.BlockSpec((4, 128, D_OUT), lambda i: (0, 0, 0)),
            pl.BlockSpec((4, 128), lambda i: (0, 0)),
            pl.BlockSpec((RB, NC), lambda i: (i, 0)),
        ],
        out_specs=[pl.BlockSpec((RB, 128), lambda i: (i, 0))] * 2,
        out_shape=[jax.ShapeDtypeStruct((N, 128), jnp.float32)] * 2,
    )(agg1, *y1_chunks, w2r, b1r, deg_t)


def _finalize(agg2, y2_chunks, b2r, deg_t):
    def body(a_r, y0, y1, b_r, d_r, o_r):
        dinv = _dinv_of(d_r[...])
        yc = (y0, y1)
        for c in range(2):
            t = a_r[0, c] + a_r[1, c] + yc[c][...]
            o_r[:, c * 128:(c + 1) * 128] = t * dinv + b_r[c][None, :]

    return pl.pallas_call(
        body,
        grid=(GRID,),
        in_specs=[
            pl.BlockSpec((NC, 2, RB, 128), lambda i: (0, 0, i, 0)),
            pl.BlockSpec((RB, 128), lambda i: (i, 0)),
            pl.BlockSpec((RB, 128), lambda i: (i, 0)),
            pl.BlockSpec((2, 128), lambda i: (0, 0)),
            pl.BlockSpec((RB, NC), lambda i: (i, 0)),
        ],
        out_specs=pl.BlockSpec((RB, D_OUT), lambda i: (i, 0)),
        out_shape=jax.ShapeDtypeStruct((N, D_OUT), jnp.float32),
    )(agg2, *y2_chunks, b2r, deg_t)


def kernel(x, edge_index, W_lin, W1, b1, W2, b2):
    src3d = edge_index[0].reshape(NW, TPT, EB)
    dst3d = edge_index[1].reshape(NW, TPT, EB)
    ones_eb = jnp.ones((EB,), jnp.float32)
    zeros_n = jnp.zeros((N2,), jnp.float32)
    zeros_blk = jnp.zeros((NPT, 128), jnp.float32)

    degp = _deg_partials(dst3d, ones_eb, zeros_n)
    deg_t = degp[:, 0, :].T
    y1c = _stage1(x, W_lin, W1, deg_t)
    agg1 = _segment_sum(y1c, src3d, dst3d, zeros_blk)
    y2c = _stage2(agg1, y1c, W2.reshape(4, 128, D_OUT),
                  b1.reshape(4, 128), deg_t)
    agg2 = _segment_sum(y2c, src3d, dst3d, zeros_blk)
    return _finalize(agg2, y2c, b2.reshape(2, 128), deg_t)


# speedup vs baseline: 1.2545x; 1.2545x over previous
import jax
import jax.numpy as jnp
from jax import lax
from jax.experimental import pallas as pl
from jax.experimental.pallas import tpu as pltpu
from jax.experimental.pallas import tpu_sc as plsc

N = 10000
N2 = 10240
E = 160000
D_IN, D_H, D_OUT = 256, 512, 256
NC, NS = 2, 16
NW = NC * NS
EB = 125
TPT = E // (NW * EB)
NPT = N2 // NS
RB = 1000
GRID = N // RB
NBUF = 2

_mesh = plsc.VectorSubcoreMesh(core_axis_name="c", subcore_axis_name="s")


def _deg_partials(dst3d, ones_eb, zeros_n):
    def body(dst_r, ones_r, zeros_r, out_r, idx_v, ones_v, acc, sem):
        core = lax.axis_index("c")
        sid = lax.axis_index("s")
        wid = core * NS + sid

        @pl.when(sid == 0)
        def _():
            pltpu.sync_copy(zeros_r, acc)

        pltpu.sync_copy(ones_r, ones_v)
        pltpu.sync_copy(dst_r.at[wid], idx_v)
        plsc.subcore_barrier()

        def step(i, carry):
            pltpu.sync_copy(ones_v, acc.at[idx_v.at[i]], add=True)
            return carry

        lax.fori_loop(0, TPT, step, 0)
        plsc.subcore_barrier()

        @pl.when(sid == 0)
        def _():
            pltpu.sync_copy(acc, out_r.at[core, 0])

    f = pl.kernel(
        body,
        out_type=jax.ShapeDtypeStruct((NC, 1, N2), jnp.float32),
        mesh=_mesh,
        scratch_types=[
            pltpu.VMEM((TPT, EB), jnp.int32),
            pltpu.VMEM((EB,), jnp.float32),
            pltpu.VMEM_SHARED((N2,), jnp.float32),
            pltpu.SemaphoreType.DMA,
        ],
    )
    return f(dst3d, ones_eb, zeros_n)


def _segment_sum(y_chunks, src3d, dst3d, zeros_blk):
    C = len(y_chunks)

    def body(*refs):
        ys = refs[:C]
        src_r, dst_r, zeros_r, out_r = refs[C:C + 4]
        src_v, dst_v = refs[C + 4:C + 6]
        rows = refs[C + 6:C + 6 + NBUF]
        acc = refs[C + 6 + NBUF]
        gsem = refs[C + 7 + NBUF:C + 7 + 2 * NBUF]
        ssem = refs[C + 7 + 2 * NBUF:]

        core = lax.axis_index("c")
        sid = lax.axis_index("s")
        wid = core * NS + sid

        pltpu.sync_copy(src_r.at[wid], src_v)
        pltpu.sync_copy(dst_r.at[wid], dst_v)

        for c in range(C):
            y_r = ys[c]

            def gather(i, b):
                pltpu.async_copy(y_r.at[src_v.at[i]], rows[b], gsem[b])

            def gather_wait(i, b):
                pltpu.make_async_copy(y_r.at[src_v.at[i]], rows[b],
                                      gsem[b]).wait()

            def scatter(i, b):
                pltpu.async_copy(rows[b], acc.at[dst_v.at[i]], ssem[b],
                                 add=True)

            def scatter_wait(i, b):
                pltpu.make_async_copy(rows[b], acc.at[dst_v.at[i]],
                                      ssem[b]).wait()

            pltpu.sync_copy(zeros_r, acc.at[pl.ds(sid * NPT, NPT)])
            plsc.subcore_barrier()

            gather(0, 0)

            def step(t, carry):
                for b in range(NBUF):
                    i = NBUF * t + b
                    nb = (b + 1) % NBUF
                    gather_wait(i, b)
                    scatter(i, b)
                    if b < NBUF - 1:
                        @pl.when(t > 0)
                        def _():
                            scatter_wait(i + 1 - NBUF, nb)

                        gather(i + 1, nb)
                    else:
                        scatter_wait(i + 1 - NBUF, nb)

                        @pl.when(t < TPT // NBUF - 1)
                        def _():
                            gather(i + 1, nb)
                return carry

            lax.fori_loop(0, TPT // NBUF, step, 0)
            for k in range(TPT - NBUF + 1, TPT):
                scatter_wait(k, k % NBUF)
            plsc.subcore_barrier()
            pltpu.sync_copy(acc.at[pl.ds(sid * NPT, NPT)],
                            out_r.at[core, c, pl.ds(sid * NPT, NPT)])
            plsc.subcore_barrier()

    f = pl.kernel(
        body,
        out_type=jax.ShapeDtypeStruct((NC, C, N2, 128), jnp.float32),
        mesh=_mesh,
        scratch_types=(
            [pltpu.VMEM((TPT, EB), jnp.int32)] * 2
            + [pltpu.VMEM((EB, 128), jnp.float32)] * NBUF
            + [pltpu.VMEM_SHARED((N2, 128), jnp.float32)]
            + [pltpu.SemaphoreType.DMA] * (2 * NBUF)
        ),
    )
    return f(*y_chunks, src3d, dst3d, zeros_blk)


def _dinv_of(d):
    return lax.rsqrt(d[:, 0:1] + d[:, 1:2] + 1.0)


def _stage1(x, w_lin, w1, deg_t):
    def body(x_r, wl_r, w1_r, d_r, *out_rs):
        wc_v = out_rs[-1]
        out_rs = out_rs[:-1]

        @pl.when(pl.program_id(0) == 0)
        def _():
            wc_v[...] = jnp.dot(wl_r[...], w1_r[...],
                                preferred_element_type=jnp.float32)

        dinv = _dinv_of(d_r[...])
        y = jnp.dot(x_r[...], wc_v[...],
                    preferred_element_type=jnp.float32) * dinv
        for c in range(4):
            out_rs[c][...] = y[:, c * 128:(c + 1) * 128]

    return pl.pallas_call(
        body,
        grid=(GRID,),
        in_specs=[
            pl.BlockSpec((RB, D_IN), lambda i: (i, 0)),
            pl.BlockSpec((D_IN, D_IN), lambda i: (0, 0)),
            pl.BlockSpec((D_IN, D_H), lambda i: (0, 0)),
            pl.BlockSpec((RB, NC), lambda i: (i, 0)),
        ],
        out_specs=[pl.BlockSpec((RB, 128), lambda i: (i, 0))] * 4,
        out_shape=[jax.ShapeDtypeStruct((N, 128), jnp.float32)] * 4,
        scratch_shapes=[pltpu.VMEM((D_IN, D_H), jnp.float32)],
    )(x, w_lin, w1, deg_t)


def _stage2(agg1, y1_chunks, w2r, b1r, deg_t):
    def body(a_r, y0, y1, y2, y3, w_r, b_r, d_r, o0, o1):
        dinv = _dinv_of(d_r[...])
        yc = (y0, y1, y2, y3)
        acc = jnp.zeros((RB, D_OUT), jnp.float32)
        for c in range(4):
            t = a_r[0, c] + a_r[1, c] + yc[c][...]
            h = jnp.maximum(t * dinv + b_r[c][None, :], 0.0)
            acc = acc + jnp.dot(h, w_r[c],
                                preferred_element_type=jnp.float32)
        y2o = acc * dinv
        o0[...] = y2o[:, :128]
        o1[...] = y2o[:, 128:]

    return pl.pallas_call(
        body,
        grid=(GRID,),
        in_specs=[
            pl.BlockSpec((NC, 4, RB, 128), lambda i: (0, 0, i, 0)),
            pl.BlockSpec((RB, 128), lambda i: (i, 0)),
            pl.BlockSpec((RB, 128), lambda i: (i, 0)),
            pl.BlockSpec((RB, 128), lambda i: (i, 0)),
            pl.BlockSpec((RB, 128), lambda i: (i, 0)),
            pl.BlockSpec((4, 128, D_OUT), lambda i: (0, 0, 0)),
            pl.BlockSpec((4, 128), lambda i: (0, 0)),
            pl.BlockSpec((RB, NC), lambda i: (i, 0)),
        ],
        out_specs=[pl.BlockSpec((RB, 128), lambda i: (i, 0))] * 2,
        out_shape=[jax.ShapeDtypeStruct((N, 128), jnp.float32)] * 2,
    )(agg1, *y1_chunks, w2r, b1r, deg_t)


def _finalize(agg2, y2_chunks, b2r, deg_t):
    def body(a_r, y0, y1, b_r, d_r, o_r):
        dinv = _dinv_of(d_r[...])
        yc = (y0, y1)
        for c in range(2):
            t = a_r[0, c] + a_r[1, c] + yc[c][...]
            o_r[:, c * 128:(c + 1) * 128] = t * dinv + b_r[c][None, :]

    return pl.pallas_call(
        body,
        grid=(GRID,),
        in_specs=[
            pl.BlockSpec((NC, 2, RB, 128), lambda i: (0, 0, i, 0)),
            pl.BlockSpec((RB, 128), lambda i: (i, 0)),
            pl.BlockSpec((RB, 128), lambda i: (i, 0)),
            pl.BlockSpec((2, 128), lambda i: (0, 0)),
            pl.BlockSpec((RB, NC), lambda i: (i, 0)),
        ],
        out_specs=pl.BlockSpec((RB, D_OUT), lambda i: (i, 0)),
        out_shape=jax.ShapeDtypeStruct((N, D_OUT), jnp.float32),
    )(agg2, *y2_chunks, b2r, deg_t)


def kernel(x, edge_index, W_lin, W1, b1, W2, b2):
    src3d = edge_index[0].reshape(NW, TPT, EB)
    dst3d = edge_index[1].reshape(NW, TPT, EB)
    ones_eb = jnp.ones((EB,), jnp.float32)
    zeros_n = jnp.zeros((N2,), jnp.float32)
    zeros_blk = jnp.zeros((NPT, 128), jnp.float32)

    degp = _deg_partials(dst3d, ones_eb, zeros_n)
    deg_t = degp[:, 0, :].T
    y1c = _stage1(x, W_lin, W1, deg_t)
    agg1 = _segment_sum(y1c, src3d, dst3d, zeros_blk)
    y2c = _stage2(agg1, y1c, W2.reshape(4, 128, D_OUT),
                  b1.reshape(4, 128), deg_t)
    agg2 = _segment_sum(y2c, src3d, dst3d, zeros_blk)
    return _finalize(agg2, y2c, b2.reshape(2, 128), deg_t)


# speedup vs baseline: 1.2562x; 1.0014x over previous
import jax
import jax.numpy as jnp
from jax import lax
from jax.experimental import pallas as pl
from jax.experimental.pallas import tpu as pltpu
from jax.experimental.pallas import tpu_sc as plsc

N = 10000
N2 = 10240
E = 160000
D_IN, D_H, D_OUT = 256, 512, 256
NC, NS = 2, 16
NW = NC * NS
EB = 125
TPT = E // (NW * EB)
NPT = N2 // NS
RB = 1000
GRID = N // RB
NBUF = 2

_mesh = plsc.VectorSubcoreMesh(core_axis_name="c", subcore_axis_name="s")


def _deg_partials(dst3d, ones_eb, zeros_n):
    def body(dst_r, ones_r, zeros_r, out_r, idx_v, ones_v, acc, sem):
        core = lax.axis_index("c")
        sid = lax.axis_index("s")
        wid = core * NS + sid

        @pl.when(sid == 0)
        def _():
            pltpu.sync_copy(zeros_r, acc)

        pltpu.sync_copy(ones_r, ones_v)
        pltpu.sync_copy(dst_r.at[wid], idx_v)
        plsc.subcore_barrier()

        def step(i, carry):
            pltpu.sync_copy(ones_v, acc.at[idx_v.at[i]], add=True)
            return carry

        lax.fori_loop(0, TPT, step, 0)
        plsc.subcore_barrier()

        @pl.when(sid == 0)
        def _():
            pltpu.sync_copy(acc, out_r.at[core, 0])

    f = pl.kernel(
        body,
        out_type=jax.ShapeDtypeStruct((NC, 1, N2), jnp.float32),
        mesh=_mesh,
        scratch_types=[
            pltpu.VMEM((TPT, EB), jnp.int32),
            pltpu.VMEM((EB,), jnp.float32),
            pltpu.VMEM_SHARED((N2,), jnp.float32),
            pltpu.SemaphoreType.DMA,
        ],
    )
    return f(dst3d, ones_eb, zeros_n)


def _segment_sum(y_chunks, src3d, dst3d, zeros_blk):
    C = len(y_chunks)

    def body(*refs):
        ys = refs[:C]
        src_r, dst_r, zeros_r, out_r = refs[C:C + 4]
        src_v, dst_v = refs[C + 4:C + 6]
        rows = refs[C + 6:C + 6 + NBUF]
        acc = refs[C + 6 + NBUF]
        gsem = refs[C + 7 + NBUF:C + 7 + 2 * NBUF]
        ssem = refs[C + 7 + 2 * NBUF:]

        core = lax.axis_index("c")
        sid = lax.axis_index("s")
        wid = core * NS + sid

        pltpu.sync_copy(src_r.at[wid], src_v)
        pltpu.sync_copy(dst_r.at[wid], dst_v)

        for c in range(C):
            y_r = ys[c]

            def gather(i, b):
                pltpu.async_copy(y_r.at[src_v.at[i]], rows[b], gsem[b])

            def gather_wait(i, b):
                pltpu.make_async_copy(y_r.at[src_v.at[i]], rows[b],
                                      gsem[b]).wait()

            def scatter(i, b):
                pltpu.async_copy(rows[b], acc.at[dst_v.at[i]], ssem[b],
                                 add=True)

            def scatter_wait(i, b):
                pltpu.make_async_copy(rows[b], acc.at[dst_v.at[i]],
                                      ssem[b]).wait()

            pltpu.sync_copy(zeros_r, acc.at[pl.ds(sid * NPT, NPT)])
            plsc.subcore_barrier()

            gather(0, 0)

            def step(t, carry):
                for b in range(NBUF):
                    i = NBUF * t + b
                    nb = (b + 1) % NBUF
                    gather_wait(i, b)
                    scatter(i, b)
                    if b < NBUF - 1:
                        @pl.when(t > 0)
                        def _():
                            scatter_wait(i + 1 - NBUF, nb)

                        gather(i + 1, nb)
                    else:
                        scatter_wait(i + 1 - NBUF, nb)

                        @pl.when(t < TPT // NBUF - 1)
                        def _():
                            gather(i + 1, nb)
                return carry

            lax.fori_loop(0, TPT // NBUF, step, 0, unroll=2)
            for k in range(TPT - NBUF + 1, TPT):
                scatter_wait(k, k % NBUF)
            plsc.subcore_barrier()
            pltpu.sync_copy(acc.at[pl.ds(sid * NPT, NPT)],
                            out_r.at[core, c, pl.ds(sid * NPT, NPT)])
            plsc.subcore_barrier()

    f = pl.kernel(
        body,
        out_type=jax.ShapeDtypeStruct((NC, C, N2, 128), jnp.float32),
        mesh=_mesh,
        scratch_types=(
            [pltpu.VMEM((TPT, EB), jnp.int32)] * 2
            + [pltpu.VMEM((EB, 128), jnp.float32)] * NBUF
            + [pltpu.VMEM_SHARED((N2, 128), jnp.float32)]
            + [pltpu.SemaphoreType.DMA] * (2 * NBUF)
        ),
    )
    return f(*y_chunks, src3d, dst3d, zeros_blk)


def _dinv_of(d):
    return lax.rsqrt(d[:, 0:1] + d[:, 1:2] + 1.0)


def _stage1(x, w_lin, w1, deg_t):
    def body(x_r, wl_r, w1_r, d_r, *out_rs):
        wc_v = out_rs[-1]
        out_rs = out_rs[:-1]

        @pl.when(pl.program_id(0) == 0)
        def _():
            wc_v[...] = jnp.dot(wl_r[...], w1_r[...],
                                preferred_element_type=jnp.float32)

        dinv = _dinv_of(d_r[...])
        y = jnp.dot(x_r[...], wc_v[...],
                    preferred_element_type=jnp.float32) * dinv
        for c in range(4):
            out_rs[c][...] = y[:, c * 128:(c + 1) * 128]

    return pl.pallas_call(
        body,
        grid=(GRID,),
        in_specs=[
            pl.BlockSpec((RB, D_IN), lambda i: (i, 0)),
            pl.BlockSpec((D_IN, D_IN), lambda i: (0, 0)),
            pl.BlockSpec((D_IN, D_H), lambda i: (0, 0)),
            pl.BlockSpec((RB, NC), lambda i: (i, 0)),
        ],
        out_specs=[pl.BlockSpec((RB, 128), lambda i: (i, 0))] * 4,
        out_shape=[jax.ShapeDtypeStruct((N, 128), jnp.float32)] * 4,
        scratch_shapes=[pltpu.VMEM((D_IN, D_H), jnp.float32)],
    )(x, w_lin, w1, deg_t)


def _stage2(agg1, y1_chunks, w2r, b1r, deg_t):
    def body(a_r, y0, y1, y2, y3, w_r, b_r, d_r, o0, o1):
        dinv = _dinv_of(d_r[...])
        yc = (y0, y1, y2, y3)
        acc = jnp.zeros((RB, D_OUT), jnp.float32)
        for c in range(4):
            t = a_r[0, c] + a_r[1, c] + yc[c][...]
            h = jnp.maximum(t * dinv + b_r[c][None, :], 0.0)
            acc = acc + jnp.dot(h, w_r[c],
                                preferred_element_type=jnp.float32)
        y2o = acc * dinv
        o0[...] = y2o[:, :128]
        o1[...] = y2o[:, 128:]

    return pl.pallas_call(
        body,
        grid=(GRID,),
        in_specs=[
            pl.BlockSpec((NC, 4, RB, 128), lambda i: (0, 0, i, 0)),
            pl.BlockSpec((RB, 128), lambda i: (i, 0)),
            pl.BlockSpec((RB, 128), lambda i: (i, 0)),
            pl.BlockSpec((RB, 128), lambda i: (i, 0)),
            pl.BlockSpec((RB, 128), lambda i: (i, 0)),
            pl.BlockSpec((4, 128, D_OUT), lambda i: (0, 0, 0)),
            pl.BlockSpec((4, 128), lambda i: (0, 0)),
            pl.BlockSpec((RB, NC), lambda i: (i, 0)),
        ],
        out_specs=[pl.BlockSpec((RB, 128), lambda i: (i, 0))] * 2,
        out_shape=[jax.ShapeDtypeStruct((N, 128), jnp.float32)] * 2,
    )(agg1, *y1_chunks, w2r, b1r, deg_t)


def _finalize(agg2, y2_chunks, b2r, deg_t):
    def body(a_r, y0, y1, b_r, d_r, o_r):
        dinv = _dinv_of(d_r[...])
        yc = (y0, y1)
        for c in range(2):
            t = a_r[0, c] + a_r[1, c] + yc[c][...]
            o_r[:, c * 128:(c + 1) * 128] = t * dinv + b_r[c][None, :]

    return pl.pallas_call(
        body,
        grid=(GRID,),
        in_specs=[
            pl.BlockSpec((NC, 2, RB, 128), lambda i: (0, 0, i, 0)),
            pl.BlockSpec((RB, 128), lambda i: (i, 0)),
            pl.BlockSpec((RB, 128), lambda i: (i, 0)),
            pl.BlockSpec((2, 128), lambda i: (0, 0)),
            pl.BlockSpec((RB, NC), lambda i: (i, 0)),
        ],
        out_specs=pl.BlockSpec((RB, D_OUT), lambda i: (i, 0)),
        out_shape=jax.ShapeDtypeStruct((N, D_OUT), jnp.float32),
    )(agg2, *y2_chunks, b2r, deg_t)


def kernel(x, edge_index, W_lin, W1, b1, W2, b2):
    src3d = edge_index[0].reshape(NW, TPT, EB)
    dst3d = edge_index[1].reshape(NW, TPT, EB)
    ones_eb = jnp.ones((EB,), jnp.float32)
    zeros_n = jnp.zeros((N2,), jnp.float32)
    zeros_blk = jnp.zeros((NPT, 128), jnp.float32)

    degp = _deg_partials(dst3d, ones_eb, zeros_n)
    deg_t = degp[:, 0, :].T
    y1c = _stage1(x, W_lin, W1, deg_t)
    agg1 = _segment_sum(y1c, src3d, dst3d, zeros_blk)
    y2c = _stage2(agg1, y1c, W2.reshape(4, 128, D_OUT),
                  b1.reshape(4, 128), deg_t)
    agg2 = _segment_sum(y2c, src3d, dst3d, zeros_blk)
    return _finalize(agg2, y2c, b2.reshape(2, 128), deg_t)
